# Initial kernel scaffold; baseline (speedup 1.0000x reference)
import jax
import jax.numpy as jnp
from jax.experimental import pallas as pl


def kernel(x, edge_index, edge_weight, gru_Wih, gru_Whh, gru_bih, gru_bhh, conv1_W, conv1_b, conv2_W, conv2_b, fc_W, fc_b):
    raise NotImplementedError("write your pallas kernel here")


# speedup vs baseline: 8.2259x; 8.2259x over previous
import functools

import jax
import jax.numpy as jnp
from jax import lax
from jax.experimental import pallas as pl
from jax.experimental.pallas import tpu as pltpu
from jax.experimental.pallas import tpu_sc as plsc

N = 50000
E = 1600000
T = 24
HID = 32
G3 = 3 * HID
OC = 16
NB = 2

GCB = 2048
NPAD2 = 49 * GCB
BN = 2000

NC, NS = 2, 16
EP = 1638400
CHUNK = 128
NHALF = N // 2
ACCROWS = 25088
RPT = ACCROWS // NS
TRASH = 25080
EDGT = EP // NS
NCHUNK = EDGT // CHUNK
EDGT_A = EP // (NC * NS)
CE = 10240

_f32 = jnp.float32


def _gru_body(x_ref, wih_ref, bih_ref, whh_ref, bhh_ref, h_ref):
    wih = wih_ref[...]
    bih = bih_ref[...]
    whh = whh_ref[...]
    bhh = bhh_ref[...]

    def step(t, h):
        xt = x_ref[pl.ds(t, 1), :]
        gi = wih * xt + bih
        gh = jnp.dot(whh, h, preferred_element_type=_f32) + bhh
        r = jax.nn.sigmoid(gi[:HID] + gh[:HID])
        z = jax.nn.sigmoid(gi[HID:2 * HID] + gh[HID:2 * HID])
        n = jnp.tanh(gi[2 * HID:] + r * gh[2 * HID:])
        return (1.0 - z) * n + z * h

    h_ref[...] = lax.fori_loop(0, T, step, jnp.zeros((HID, GCB), _f32))


def _gru_call(x_fm, wih, bih, whh, bhh):
    return pl.pallas_call(
        _gru_body,
        grid=(NPAD2 // GCB,),
        in_specs=[
            pl.BlockSpec((T, GCB), lambda i: (0, i)),
            pl.BlockSpec((G3, 1), lambda i: (0, 0)),
            pl.BlockSpec((G3, 1), lambda i: (0, 0)),
            pl.BlockSpec((G3, HID), lambda i: (0, 0)),
            pl.BlockSpec((G3, 1), lambda i: (0, 0)),
        ],
        out_specs=pl.BlockSpec((HID, GCB), lambda i: (0, i)),
        out_shape=jax.ShapeDtypeStruct((HID, NPAD2), _f32),
    )(x_fm, wih, bih, whh, bhh)


def _deg_body(col_hbm, ew_hbm, out_hbm, deg_v, col_v, ew_v):
    c = lax.axis_index("c")
    s = lax.axis_index("s")
    g = c * NS + s
    zero16 = jnp.zeros((16,), _f32)

    def zbody(i, carry):
        deg_v[pl.ds(i * 16, 16)] = zero16
        return carry

    lax.fori_loop(0, N // 16, zbody, 0)

    base = g * EDGT_A

    def chunk(k, carry):
        pltpu.sync_copy(col_hbm.at[pl.ds(base + k * CE, CE)], col_v)
        pltpu.sync_copy(ew_hbm.at[pl.ds(base + k * CE, CE)], ew_v)

        def inner(j, cc):
            idx = col_v[pl.ds(j * 16, 16)]
            w = ew_v[pl.ds(j * 16, 16)]
            plsc.addupdate_scatter(deg_v, [idx], w)
            return cc

        lax.fori_loop(0, CE // 16, inner, 0)
        return carry

    lax.fori_loop(0, EDGT_A // CE, chunk, 0)
    pltpu.sync_copy(deg_v, out_hbm.at[g])


def _deg_call(colp, ewp):
    fn = functools.partial(
        pl.kernel,
        out_type=jax.ShapeDtypeStruct((NC * NS, N), _f32),
        mesh=plsc.VectorSubcoreMesh(
            core_axis_name="c", subcore_axis_name="s",
            num_cores=NC, num_subcores=NS),
        scratch_types=[
            pltpu.VMEM((N,), _f32),
            pltpu.VMEM((CE,), jnp.int32),
            pltpu.VMEM((CE,), _f32),
        ],
        compiler_params=pltpu.CompilerParams(
            needs_layout_passes=False, use_tc_tiling_on_sc=False),
    )(_deg_body)
    return fn(colp, ewp)


def _make_agg_body(width):
    def agg_body(xw_hbm, row_hbm, col_hbm, ew_hbm, zeros_hbm, out_hbm,
                 acc, row_v, idx_v, ew_v, rows_v, sem):
        c = lax.axis_index("c")
        s = lax.axis_index("s")
        nbase = c * NHALF

        pltpu.sync_copy(zeros_hbm, acc.at[pl.ds(s * RPT, RPT)])
        plsc.subcore_barrier()

        ebase = s * EDGT

        def chunk(k, carry):
            eb = ebase + k * CHUNK
            pltpu.sync_copy(row_hbm.at[pl.ds(eb, CHUNK)], row_v)
            pltpu.sync_copy(col_hbm.at[pl.ds(eb, CHUNK)], idx_v)
            pltpu.sync_copy(ew_hbm.at[pl.ds(eb, CHUNK)], ew_v)
            pltpu.async_copy(xw_hbm.at[row_v], rows_v, sem).wait()
            for j8 in range(CHUNK // 16):
                v = idx_v[pl.ds(j8 * 16, 16)] - nbase
                inr = (v >= 0) & (v < NHALF)
                idx_v[pl.ds(j8 * 16, 16)] = jnp.where(inr, v, TRASH)

            def scale(j16, cc):
                w16 = ew_v[pl.ds(j16 * 16, 16)]
                for l in range(16):
                    w = w16[l]
                    j = j16 * 16 + l
                    for q in range(width // 16):
                        rows_v[j, pl.ds(q * 16, 16)] = (
                            rows_v[j, pl.ds(q * 16, 16)] * w)
                return cc

            lax.fori_loop(0, CHUNK // 16, scale, 0)
            pltpu.sync_copy(rows_v, acc.at[idx_v], add=True)
            return carry

        lax.fori_loop(0, NCHUNK, chunk, 0)
        plsc.subcore_barrier()
        pltpu.sync_copy(acc.at[pl.ds(s * RPT, RPT)],
                        out_hbm.at[pl.ds(c * ACCROWS + s * RPT, RPT)])

    return agg_body


def _agg_call(width, xw, rowp, colp, ewp, zrows):
    fn = functools.partial(
        pl.kernel,
        out_type=jax.ShapeDtypeStruct((NC * ACCROWS, width), _f32),
        mesh=plsc.VectorSubcoreMesh(
            core_axis_name="c", subcore_axis_name="s",
            num_cores=NC, num_subcores=NS),
        scratch_types=[
            pltpu.VMEM_SHARED((ACCROWS, width), _f32),
            pltpu.VMEM((CHUNK,), jnp.int32),
            pltpu.VMEM((CHUNK,), jnp.int32),
            pltpu.VMEM((CHUNK,), _f32),
            pltpu.VMEM((CHUNK, width), _f32),
            pltpu.SemaphoreType.DMA,
        ],
        compiler_params=pltpu.CompilerParams(
            needs_layout_passes=False, use_tc_tiling_on_sc=False),
    )(_make_agg_body(width))
    return fn(xw, rowp, colp, ewp, zrows)


def _mix1_body(h0_ref, h1_ref, degT_ref, w1_ref, xw_ref, dinv_ref):
    d = jnp.sum(degT_ref[...], axis=1, keepdims=True) + 1.0
    dinv = lax.rsqrt(d)
    w1 = w1_ref[...]
    a0 = jnp.dot(h0_ref[...], w1, preferred_element_type=_f32)
    a1 = jnp.dot(h1_ref[...], w1, preferred_element_type=_f32)
    xw_ref[...] = jnp.concatenate([a0, a1], axis=1) * dinv
    dinv_ref[...] = dinv


def _mix1_call(h0, h1, degT, w1):
    return pl.pallas_call(
        _mix1_body,
        grid=(N // BN,),
        in_specs=[
            pl.BlockSpec((BN, HID), lambda i: (i, 0)),
            pl.BlockSpec((BN, HID), lambda i: (i, 0)),
            pl.BlockSpec((BN, NC * NS), lambda i: (i, 0)),
            pl.BlockSpec((HID, HID), lambda i: (0, 0)),
        ],
        out_specs=[
            pl.BlockSpec((BN, 2 * HID), lambda i: (i, 0)),
            pl.BlockSpec((BN, 1), lambda i: (i, 0)),
        ],
        out_shape=[
            jax.ShapeDtypeStruct((N, 2 * HID), _f32),
            jax.ShapeDtypeStruct((N, 1), _f32),
        ],
    )(h0, h1, degT, w1)


def _mix2_body(agg_ref, xwp_ref, dinv_ref, b1t_ref, w2bd_ref, out_ref):
    dinv = dinv_ref[...]
    y = jnp.maximum(dinv * (agg_ref[...] + xwp_ref[...]) + b1t_ref[...], 0.0)
    out_ref[...] = jnp.dot(y, w2bd_ref[...], preferred_element_type=_f32) * dinv


def _mix2_call(agg1, xw1p, dinv, b1t, w2bd):
    return pl.pallas_call(
        _mix2_body,
        grid=(N // BN,),
        in_specs=[
            pl.BlockSpec((BN, 2 * HID), lambda i: (i, 0)),
            pl.BlockSpec((BN, 2 * HID), lambda i: (i, 0)),
            pl.BlockSpec((BN, 1), lambda i: (i, 0)),
            pl.BlockSpec((1, 2 * HID), lambda i: (0, 0)),
            pl.BlockSpec((2 * HID, 2 * OC), lambda i: (0, 0)),
        ],
        out_specs=pl.BlockSpec((BN, 2 * OC), lambda i: (i, 0)),
        out_shape=jax.ShapeDtypeStruct((N, 2 * OC), _f32),
    )(agg1, xw1p, dinv, b1t, w2bd)


def _fin_body(agg_ref, xwp_ref, dinv_ref, b2t_ref, wf_ref, out_ref):
    y = dinv_ref[...] * (agg_ref[...] + xwp_ref[...]) + b2t_ref[...]
    out_ref[...] = jnp.dot(y, wf_ref[...], preferred_element_type=_f32)


def _fin_call(agg2, xw2p, dinv, b2t, wf):
    return pl.pallas_call(
        _fin_body,
        grid=(N // BN,),
        in_specs=[
            pl.BlockSpec((BN, 2 * OC), lambda i: (i, 0)),
            pl.BlockSpec((BN, 2 * OC), lambda i: (i, 0)),
            pl.BlockSpec((BN, 1), lambda i: (i, 0)),
            pl.BlockSpec((1, 2 * OC), lambda i: (0, 0)),
            pl.BlockSpec((2 * OC, NB), lambda i: (0, 0)),
        ],
        out_specs=pl.BlockSpec((BN, NB), lambda i: (i, 0)),
        out_shape=jax.ShapeDtypeStruct((N, NB), _f32),
    )(agg2, xw2p, dinv, b2t, wf)


def kernel(x, edge_index, edge_weight, gru_Wih, gru_Whh, gru_bih, gru_bhh,
           conv1_W, conv1_b, conv2_W, conv2_b, fc_W, fc_b):
    x_fm = x.transpose(2, 0, 1).reshape(T, NB * N)
    x_fm = jnp.pad(x_fm, ((0, 0), (0, NPAD2 - NB * N)))
    h_fm = _gru_call(x_fm, gru_Wih[:, 0:1], gru_bih[:, None],
                     gru_Whh, gru_bhh[:, None])
    h0 = h_fm[:, :N].T
    h1 = h_fm[:, N:2 * N].T

    row = edge_index[0]
    col = edge_index[1]
    rowp = jnp.pad(row, (0, EP - E))
    colp = jnp.pad(col, (0, EP - E))
    ewp = jnp.pad(edge_weight, (0, EP - E))

    deg_parts = _deg_call(colp, ewp)
    degT = deg_parts.T

    xw1p, dinv = _mix1_call(h0, h1, degT, conv1_W)

    z64 = jnp.zeros((RPT, 2 * HID), _f32)
    agg1p = _agg_call(2 * HID, xw1p, rowp, colp, ewp, z64)
    agg1 = agg1p.reshape(NC, ACCROWS, 2 * HID)[:, :NHALF].reshape(N, 2 * HID)

    b1t = jnp.concatenate([conv1_b, conv1_b])[None, :]
    w2bd = jnp.zeros((2 * HID, 2 * OC), _f32)
    w2bd = w2bd.at[:HID, :OC].set(conv2_W).at[HID:, OC:].set(conv2_W)
    xw2p = _mix2_call(agg1, xw1p, dinv, b1t, w2bd)

    z32 = jnp.zeros((RPT, 2 * OC), _f32)
    agg2p = _agg_call(2 * OC, xw2p, rowp, colp, ewp, z32)
    agg2 = agg2p.reshape(NC, ACCROWS, 2 * OC)[:, :NHALF].reshape(N, 2 * OC)

    b2t = jnp.concatenate([conv2_b, conv2_b])[None, :]
    wf = jnp.zeros((2 * OC, NB), _f32)
    wf = wf.at[:OC, 0].set(fc_W[0]).at[OC:, 1].set(fc_W[0])
    y = _fin_call(agg2, xw2p, dinv, b2t, wf)

    return (y + fc_b).T


# speedup vs baseline: 14.2140x; 1.7279x over previous
import functools

import jax
import jax.numpy as jnp
from jax import lax
from jax.experimental import pallas as pl
from jax.experimental.pallas import tpu as pltpu
from jax.experimental.pallas import tpu_sc as plsc

N = 50000
E = 1600000
T = 24
HID = 32
G3 = 3 * HID
OC = 16
NB = 2

GCB = 2048
NPAD2 = 49 * GCB
BN = 2000

NC, NS = 2, 16
EP = 1638400
CHUNK = 128
NHALF = N // 2
ACCROWS = 25088
RPT = ACCROWS // NS
TRASH = 25080
EDGT = EP // NS
NCHT = EDGT // CHUNK
BLK = 32
NBLK = NCHT // BLK
EDGT_A = EP // (NC * NS)
CE = 10240

_f32 = jnp.float32


def _gru_body(x_ref, wih_ref, bih_ref, whh_ref, bhh_ref, h_ref):
    wih = wih_ref[...]
    bih = bih_ref[...]
    whh = whh_ref[...]
    bhh = bhh_ref[...]

    def step(t, h):
        xt = x_ref[pl.ds(t, 1), :]
        gi = wih * xt + bih
        gh = jnp.dot(whh, h, preferred_element_type=_f32) + bhh
        r = jax.nn.sigmoid(gi[:HID] + gh[:HID])
        z = jax.nn.sigmoid(gi[HID:2 * HID] + gh[HID:2 * HID])
        n = jnp.tanh(gi[2 * HID:] + r * gh[2 * HID:])
        return (1.0 - z) * n + z * h

    h_ref[...] = lax.fori_loop(0, T, step, jnp.zeros((HID, GCB), _f32))


def _gru_call(x_fm, wih, bih, whh, bhh):
    return pl.pallas_call(
        _gru_body,
        grid=(NPAD2 // GCB,),
        in_specs=[
            pl.BlockSpec((T, GCB), lambda i: (0, i)),
            pl.BlockSpec((G3, 1), lambda i: (0, 0)),
            pl.BlockSpec((G3, 1), lambda i: (0, 0)),
            pl.BlockSpec((G3, HID), lambda i: (0, 0)),
            pl.BlockSpec((G3, 1), lambda i: (0, 0)),
        ],
        out_specs=pl.BlockSpec((HID, GCB), lambda i: (0, i)),
        out_shape=jax.ShapeDtypeStruct((HID, NPAD2), _f32),
    )(x_fm, wih, bih, whh, bhh)


def _deg_body(col_hbm, ew_hbm, out_hbm, deg_v, col_v, ew_v):
    c = lax.axis_index("c")
    s = lax.axis_index("s")
    g = c * NS + s
    zero16 = jnp.zeros((16,), _f32)

    def zbody(i, carry):
        deg_v[pl.ds(i * 16, 16)] = zero16
        return carry

    lax.fori_loop(0, N // 16, zbody, 0)

    base = g * EDGT_A

    def chunk(k, carry):
        pltpu.sync_copy(col_hbm.at[pl.ds(base + k * CE, CE)], col_v)
        pltpu.sync_copy(ew_hbm.at[pl.ds(base + k * CE, CE)], ew_v)

        def inner(j, cc):
            idx = col_v[pl.ds(j * 16, 16)]
            w = ew_v[pl.ds(j * 16, 16)]
            plsc.addupdate_scatter(deg_v, [idx], w)
            return cc

        lax.fori_loop(0, CE // 16, inner, 0)
        return carry

    lax.fori_loop(0, EDGT_A // CE, chunk, 0)
    pltpu.sync_copy(deg_v, out_hbm.at[g])


def _deg_call(colp, ewp):
    fn = functools.partial(
        pl.kernel,
        out_type=jax.ShapeDtypeStruct((NC * NS, N), _f32),
        mesh=plsc.VectorSubcoreMesh(
            core_axis_name="c", subcore_axis_name="s",
            num_cores=NC, num_subcores=NS),
        scratch_types=[
            pltpu.VMEM((N,), _f32),
            pltpu.VMEM((CE,), jnp.int32),
            pltpu.VMEM((CE,), _f32),
        ],
        compiler_params=pltpu.CompilerParams(
            needs_layout_passes=False, use_tc_tiling_on_sc=False),
    )(_deg_body)
    return fn(colp, ewp)


def _make_agg_body(width):
    def agg_body(xw_hbm, row_hbm, col_hbm, ew_hbm, zeros_hbm, out_hbm,
                 acc, rowblk, colblk, ewblk, rows0, rows1, gsem0, gsem1):
        c_ax = lax.axis_index("c")
        s = lax.axis_index("s")
        nbase = c_ax * NHALF

        pltpu.sync_copy(zeros_hbm, acc.at[pl.ds(s * RPT, RPT)])
        plsc.subcore_barrier()

        tbase = s * NCHT

        def blk_body(bi, carry):
            cb = tbase + bi * BLK
            pltpu.sync_copy(row_hbm.at[pl.ds(cb, BLK)], rowblk)
            pltpu.sync_copy(col_hbm.at[pl.ds(cb, BLK)], colblk)
            pltpu.sync_copy(ew_hbm.at[pl.ds(cb, BLK)], ewblk)
            pltpu.async_copy(xw_hbm.at[rowblk.at[0]], rows0, gsem0)

            def grp(g, cc):
                for l in range(2):
                    c = g * 2 + l
                    if l == 0:
                        buf, gsem, nbuf, ngsem = rows0, gsem0, rows1, gsem1
                    else:
                        buf, gsem, nbuf, ngsem = rows1, gsem1, rows0, gsem0

                    @pl.when(c + 1 < BLK)
                    def _():
                        pltpu.async_copy(
                            xw_hbm.at[rowblk.at[c + 1]], nbuf, ngsem)

                    pltpu.make_async_copy(
                        xw_hbm.at[rowblk.at[c]], buf, gsem).wait()
                    for j8 in range(CHUNK // 16):
                        v = colblk[c, pl.ds(j8 * 16, 16)] - nbase
                        inr = (v >= 0) & (v < NHALF)
                        colblk[c, pl.ds(j8 * 16, 16)] = jnp.where(
                            inr, v, TRASH)
                    for j16 in range(CHUNK // 16):
                        w16 = ewblk[c, pl.ds(j16 * 16, 16)]
                        for ll in range(16):
                            w = w16[ll]
                            j = j16 * 16 + ll
                            for q in range(width // 16):
                                buf[j, pl.ds(q * 16, 16)] = (
                                    buf[j, pl.ds(q * 16, 16)] * w)
                    pltpu.sync_copy(buf, acc.at[colblk.at[c]], add=True)
                return cc

            lax.fori_loop(0, BLK // 2, grp, 0)
            return carry

        lax.fori_loop(0, NBLK, blk_body, 0)
        plsc.subcore_barrier()
        pltpu.sync_copy(acc.at[pl.ds(s * RPT, RPT)],
                        out_hbm.at[pl.ds(c_ax * ACCROWS + s * RPT, RPT)])

    return agg_body


def _agg_call(width, xw, row2, col2, ew2, zrows):
    fn = functools.partial(
        pl.kernel,
        out_type=jax.ShapeDtypeStruct((NC * ACCROWS, width), _f32),
        mesh=plsc.VectorSubcoreMesh(
            core_axis_name="c", subcore_axis_name="s",
            num_cores=NC, num_subcores=NS),
        scratch_types=[
            pltpu.VMEM_SHARED((ACCROWS, width), _f32),
            pltpu.VMEM((BLK, CHUNK), jnp.int32),
            pltpu.VMEM((BLK, CHUNK), jnp.int32),
            pltpu.VMEM((BLK, CHUNK), _f32),
            pltpu.VMEM((CHUNK, width), _f32),
            pltpu.VMEM((CHUNK, width), _f32),
            pltpu.SemaphoreType.DMA,
            pltpu.SemaphoreType.DMA,
        ],
        compiler_params=pltpu.CompilerParams(
            needs_layout_passes=False, use_tc_tiling_on_sc=False),
    )(_make_agg_body(width))
    return fn(xw, row2, col2, ew2, zrows)


def _mix1_body(h0_ref, h1_ref, degT_ref, w1_ref, xw_ref, dinv_ref):
    d = jnp.sum(degT_ref[...], axis=1, keepdims=True) + 1.0
    dinv = lax.rsqrt(d)
    w1 = w1_ref[...]
    a0 = jnp.dot(h0_ref[...], w1, preferred_element_type=_f32)
    a1 = jnp.dot(h1_ref[...], w1, preferred_element_type=_f32)
    xw_ref[...] = jnp.concatenate([a0, a1], axis=1) * dinv
    dinv_ref[...] = dinv


def _mix1_call(h0, h1, degT, w1):
    return pl.pallas_call(
        _mix1_body,
        grid=(N // BN,),
        in_specs=[
            pl.BlockSpec((BN, HID), lambda i: (i, 0)),
            pl.BlockSpec((BN, HID), lambda i: (i, 0)),
            pl.BlockSpec((BN, NC * NS), lambda i: (i, 0)),
            pl.BlockSpec((HID, HID), lambda i: (0, 0)),
        ],
        out_specs=[
            pl.BlockSpec((BN, 2 * HID), lambda i: (i, 0)),
            pl.BlockSpec((BN, 1), lambda i: (i, 0)),
        ],
        out_shape=[
            jax.ShapeDtypeStruct((N, 2 * HID), _f32),
            jax.ShapeDtypeStruct((N, 1), _f32),
        ],
    )(h0, h1, degT, w1)


def _mix2_body(agg_ref, xwp_ref, dinv_ref, b1t_ref, w2bd_ref, out_ref):
    dinv = dinv_ref[...]
    y = jnp.maximum(dinv * (agg_ref[...] + xwp_ref[...]) + b1t_ref[...], 0.0)
    out_ref[...] = jnp.dot(y, w2bd_ref[...], preferred_element_type=_f32) * dinv


def _mix2_call(agg1, xw1p, dinv, b1t, w2bd):
    return pl.pallas_call(
        _mix2_body,
        grid=(N // BN,),
        in_specs=[
            pl.BlockSpec((BN, 2 * HID), lambda i: (i, 0)),
            pl.BlockSpec((BN, 2 * HID), lambda i: (i, 0)),
            pl.BlockSpec((BN, 1), lambda i: (i, 0)),
            pl.BlockSpec((1, 2 * HID), lambda i: (0, 0)),
            pl.BlockSpec((2 * HID, 2 * OC), lambda i: (0, 0)),
        ],
        out_specs=pl.BlockSpec((BN, 2 * OC), lambda i: (i, 0)),
        out_shape=jax.ShapeDtypeStruct((N, 2 * OC), _f32),
    )(agg1, xw1p, dinv, b1t, w2bd)


def _fin_body(agg_ref, xwp_ref, dinv_ref, b2t_ref, wf_ref, out_ref):
    y = dinv_ref[...] * (agg_ref[...] + xwp_ref[...]) + b2t_ref[...]
    out_ref[...] = jnp.dot(y, wf_ref[...], preferred_element_type=_f32)


def _fin_call(agg2, xw2p, dinv, b2t, wf):
    return pl.pallas_call(
        _fin_body,
        grid=(N // BN,),
        in_specs=[
            pl.BlockSpec((BN, 2 * OC), lambda i: (i, 0)),
            pl.BlockSpec((BN, 2 * OC), lambda i: (i, 0)),
            pl.BlockSpec((BN, 1), lambda i: (i, 0)),
            pl.BlockSpec((1, 2 * OC), lambda i: (0, 0)),
            pl.BlockSpec((2 * OC, NB), lambda i: (0, 0)),
        ],
        out_specs=pl.BlockSpec((BN, NB), lambda i: (i, 0)),
        out_shape=jax.ShapeDtypeStruct((N, NB), _f32),
    )(agg2, xw2p, dinv, b2t, wf)


def kernel(x, edge_index, edge_weight, gru_Wih, gru_Whh, gru_bih, gru_bhh,
           conv1_W, conv1_b, conv2_W, conv2_b, fc_W, fc_b):
    x_fm = x.transpose(2, 0, 1).reshape(T, NB * N)
    x_fm = jnp.pad(x_fm, ((0, 0), (0, NPAD2 - NB * N)))
    h_fm = _gru_call(x_fm, gru_Wih[:, 0:1], gru_bih[:, None],
                     gru_Whh, gru_bhh[:, None])
    h0 = h_fm[:, :N].T
    h1 = h_fm[:, N:2 * N].T

    row = edge_index[0]
    col = edge_index[1]
    rowp = jnp.pad(row, (0, EP - E))
    colp = jnp.pad(col, (0, EP - E))
    ewp = jnp.pad(edge_weight, (0, EP - E))

    deg_parts = _deg_call(colp, ewp)
    degT = deg_parts.T

    xw1p, dinv = _mix1_call(h0, h1, degT, conv1_W)

    row2 = rowp.reshape(EP // CHUNK, CHUNK)
    col2 = colp.reshape(EP // CHUNK, CHUNK)
    ew2 = ewp.reshape(EP // CHUNK, CHUNK)

    z64 = jnp.zeros((RPT, 2 * HID), _f32)
    agg1p = _agg_call(2 * HID, xw1p, row2, col2, ew2, z64)
    agg1 = agg1p.reshape(NC, ACCROWS, 2 * HID)[:, :NHALF].reshape(N, 2 * HID)

    b1t = jnp.concatenate([conv1_b, conv1_b])[None, :]
    w2bd = jnp.zeros((2 * HID, 2 * OC), _f32)
    w2bd = w2bd.at[:HID, :OC].set(conv2_W).at[HID:, OC:].set(conv2_W)
    xw2p = _mix2_call(agg1, xw1p, dinv, b1t, w2bd)

    z32 = jnp.zeros((RPT, 2 * OC), _f32)
    agg2p = _agg_call(2 * OC, xw2p, row2, col2, ew2, z32)
    agg2 = agg2p.reshape(NC, ACCROWS, 2 * OC)[:, :NHALF].reshape(N, 2 * OC)

    b2t = jnp.concatenate([conv2_b, conv2_b])[None, :]
    wf = jnp.zeros((2 * OC, NB), _f32)
    wf = wf.at[:OC, 0].set(fc_W[0]).at[OC:, 1].set(fc_W[0])
    y = _fin_call(agg2, xw2p, dinv, b2t, wf)

    return (y + fc_b).T


# speedup vs baseline: 16.8745x; 1.1872x over previous
import functools

import jax
import jax.numpy as jnp
from jax import lax
from jax.experimental import pallas as pl
from jax.experimental.pallas import tpu as pltpu
from jax.experimental.pallas import tpu_sc as plsc

N = 50000
E = 1600000
T = 24
HID = 32
G3 = 3 * HID
OC = 16
NB = 2

GCB = 2048
NPAD2 = 49 * GCB
BN = 2000

NC, NS = 2, 16
CHUNK = 128
BLK = 12
EP = 1622016
NHALF = N // 2
ACC1 = 25008
RPT1 = ACC1 // NS
TRASH = 25000
NBLK1 = 66
ACC2 = 50048
RPT2 = ACC2 // NS
NBLK2 = 33
EDGT_A = EP // (NC * NS)
CE = 6336

_f32 = jnp.float32


def _gru_body(x_ref, wih_ref, bih_ref, whh_ref, bhh_ref, h_ref):
    wih = wih_ref[...]
    bih = bih_ref[...]
    whh = whh_ref[...]
    bhh = bhh_ref[...]

    def step(t, h):
        xt = x_ref[pl.ds(t, 1), :]
        gi = wih * xt + bih
        gh = jnp.dot(whh, h, preferred_element_type=_f32) + bhh
        r = jax.nn.sigmoid(gi[:HID] + gh[:HID])
        z = jax.nn.sigmoid(gi[HID:2 * HID] + gh[HID:2 * HID])
        n = jnp.tanh(gi[2 * HID:] + r * gh[2 * HID:])
        return (1.0 - z) * n + z * h

    h_ref[...] = lax.fori_loop(0, T, step, jnp.zeros((HID, GCB), _f32))


def _gru_call(x_fm, wih, bih, whh, bhh):
    return pl.pallas_call(
        _gru_body,
        grid=(NPAD2 // GCB,),
        in_specs=[
            pl.BlockSpec((T, GCB), lambda i: (0, i)),
            pl.BlockSpec((G3, 1), lambda i: (0, 0)),
            pl.BlockSpec((G3, 1), lambda i: (0, 0)),
            pl.BlockSpec((G3, HID), lambda i: (0, 0)),
            pl.BlockSpec((G3, 1), lambda i: (0, 0)),
        ],
        out_specs=pl.BlockSpec((HID, GCB), lambda i: (0, i)),
        out_shape=jax.ShapeDtypeStruct((HID, NPAD2), _f32),
    )(x_fm, wih, bih, whh, bhh)


def _deg_body(col_hbm, ew_hbm, out_hbm, deg_v, col_v, ew_v):
    c = lax.axis_index("c")
    s = lax.axis_index("s")
    g = c * NS + s
    zero16 = jnp.zeros((16,), _f32)

    def zbody(i, carry):
        deg_v[pl.ds(i * 16, 16)] = zero16
        return carry

    lax.fori_loop(0, N // 16, zbody, 0)

    base = g * EDGT_A

    def chunk(k, carry):
        pltpu.sync_copy(col_hbm.at[pl.ds(base + k * CE, CE)], col_v)
        pltpu.sync_copy(ew_hbm.at[pl.ds(base + k * CE, CE)], ew_v)

        def inner(j, cc):
            idx = col_v[pl.ds(j * 16, 16)]
            w = ew_v[pl.ds(j * 16, 16)]
            plsc.addupdate_scatter(deg_v, [idx], w)
            return cc

        lax.fori_loop(0, CE // 16, inner, 0)
        return carry

    lax.fori_loop(0, EDGT_A // CE, chunk, 0)
    pltpu.sync_copy(deg_v, out_hbm.at[g])


def _deg_call(colp, ewp):
    fn = functools.partial(
        pl.kernel,
        out_type=jax.ShapeDtypeStruct((NC * NS, N), _f32),
        mesh=plsc.VectorSubcoreMesh(
            core_axis_name="c", subcore_axis_name="s",
            num_cores=NC, num_subcores=NS),
        scratch_types=[
            pltpu.VMEM((N,), _f32),
            pltpu.VMEM((CE,), jnp.int32),
            pltpu.VMEM((CE,), _f32),
        ],
        compiler_params=pltpu.CompilerParams(
            needs_layout_passes=False, use_tc_tiling_on_sc=False),
    )(_deg_body)
    return fn(colp, ewp)


def _make_agg_body(width, acc_rows, rpt, nblk, dst_split):
    def agg_body(xw_hbm, row_hbm, col_hbm, ew_hbm, zeros_hbm, out_hbm,
                 acc, rowblk, colblk, ewblk, r0, r1, r2,
                 g0, g1, g2, s0, s1, s2):
        c_ax = lax.axis_index("c")
        s = lax.axis_index("s")
        rows = [r0, r1, r2]
        gsems = [g0, g1, g2]
        ssems = [s0, s1, s2]

        pltpu.sync_copy(zeros_hbm, acc.at[pl.ds(s * rpt, rpt)])
        plsc.subcore_barrier()

        if dst_split:
            tbase = s * (nblk * BLK)
        else:
            tbase = (c_ax * NS + s) * (nblk * BLK)
        nbase = c_ax * NHALF

        def gfire(ci, b):
            pltpu.async_copy(xw_hbm.at[rowblk.at[ci]], rows[b], gsems[b])

        def gwait(ci, b):
            pltpu.make_async_copy(
                xw_hbm.at[rowblk.at[ci]], rows[b], gsems[b]).wait()

        def sfire(ci, b):
            pltpu.async_copy(rows[b], acc.at[colblk.at[ci]], ssems[b],
                             add=True)

        def swait(b):
            pltpu.make_async_copy(
                rows[b], acc.at[colblk.at[0]], ssems[b]).wait()

        def blk_body(bi, carry):
            @pl.when(bi > 0)
            def _():
                for l in range(3):
                    swait(l)

            cb = tbase + bi * BLK
            pltpu.sync_copy(row_hbm.at[pl.ds(cb, BLK)], rowblk)
            pltpu.sync_copy(col_hbm.at[pl.ds(cb, BLK)], colblk)
            pltpu.sync_copy(ew_hbm.at[pl.ds(cb, BLK)], ewblk)
            for l in range(2):
                gfire(l, l)

            def grp(g, cc):
                for l in range(3):
                    c = g * 3 + l
                    gwait(c, l)
                    if dst_split:
                        for j8 in range(CHUNK // 16):
                            v = colblk[c, pl.ds(j8 * 16, 16)] - nbase
                            inr = (v >= 0) & (v < NHALF)
                            colblk[c, pl.ds(j8 * 16, 16)] = jnp.where(
                                inr, v, TRASH)

                    def scale(j16, sc_c):
                        w16 = ewblk[c, pl.ds(j16 * 16, 16)]
                        for ll in range(16):
                            w = w16[ll]
                            j = j16 * 16 + ll
                            for q in range(width // 16):
                                rows[l][j, pl.ds(q * 16, 16)] = (
                                    rows[l][j, pl.ds(q * 16, 16)] * w)
                        return sc_c

                    lax.fori_loop(0, CHUNK // 16, scale, 0)
                    sfire(c, l)
                    nb = (l + 2) % 3

                    @pl.when(c + 2 < BLK)
                    def _():
                        @pl.when(c > 0)
                        def _():
                            swait(nb)
                        gfire(c + 2, nb)
                return cc

            lax.fori_loop(0, BLK // 3, grp, 0)
            return carry

        lax.fori_loop(0, nblk, blk_body, 0)
        for l in range(3):
            swait(l)
        plsc.subcore_barrier()
        pltpu.sync_copy(acc.at[pl.ds(s * rpt, rpt)],
                        out_hbm.at[pl.ds(c_ax * acc_rows + s * rpt, rpt)])

    return agg_body


def _agg_call(width, acc_rows, rpt, nblk, dst_split, xw, row2, col2, ew2,
              zrows):
    fn = functools.partial(
        pl.kernel,
        out_type=jax.ShapeDtypeStruct((NC * acc_rows, width), _f32),
        mesh=plsc.VectorSubcoreMesh(
            core_axis_name="c", subcore_axis_name="s",
            num_cores=NC, num_subcores=NS),
        scratch_types=[
            pltpu.VMEM_SHARED((acc_rows, width), _f32),
            pltpu.VMEM((BLK, CHUNK), jnp.int32),
            pltpu.VMEM((BLK, CHUNK), jnp.int32),
            pltpu.VMEM((BLK, CHUNK), _f32),
            pltpu.VMEM((CHUNK, width), _f32),
            pltpu.VMEM((CHUNK, width), _f32),
            pltpu.VMEM((CHUNK, width), _f32),
            pltpu.SemaphoreType.DMA,
            pltpu.SemaphoreType.DMA,
            pltpu.SemaphoreType.DMA,
            pltpu.SemaphoreType.DMA,
            pltpu.SemaphoreType.DMA,
            pltpu.SemaphoreType.DMA,
        ],
        compiler_params=pltpu.CompilerParams(
            needs_layout_passes=False, use_tc_tiling_on_sc=False),
    )(_make_agg_body(width, acc_rows, rpt, nblk, dst_split))
    return fn(xw, row2, col2, ew2, zrows)


def _mix1_body(h0_ref, h1_ref, degT_ref, w1_ref, xw_ref, dinv_ref):
    d = jnp.sum(degT_ref[...], axis=1, keepdims=True) + 1.0
    dinv = lax.rsqrt(d)
    w1 = w1_ref[...]
    a0 = jnp.dot(h0_ref[...], w1, preferred_element_type=_f32)
    a1 = jnp.dot(h1_ref[...], w1, preferred_element_type=_f32)
    xw_ref[...] = jnp.concatenate([a0, a1], axis=1) * dinv
    dinv_ref[...] = dinv


def _mix1_call(h0, h1, degT, w1):
    return pl.pallas_call(
        _mix1_body,
        grid=(N // BN,),
        in_specs=[
            pl.BlockSpec((BN, HID), lambda i: (i, 0)),
            pl.BlockSpec((BN, HID), lambda i: (i, 0)),
            pl.BlockSpec((BN, NC * NS), lambda i: (i, 0)),
            pl.BlockSpec((HID, HID), lambda i: (0, 0)),
        ],
        out_specs=[
            pl.BlockSpec((BN, 2 * HID), lambda i: (i, 0)),
            pl.BlockSpec((BN, 1), lambda i: (i, 0)),
        ],
        out_shape=[
            jax.ShapeDtypeStruct((N, 2 * HID), _f32),
            jax.ShapeDtypeStruct((N, 1), _f32),
        ],
    )(h0, h1, degT, w1)


def _mix2_body(agg_ref, xwp_ref, dinv_ref, b1t_ref, w2bd_ref, out_ref):
    dinv = dinv_ref[...]
    y = jnp.maximum(dinv * (agg_ref[...] + xwp_ref[...]) + b1t_ref[...], 0.0)
    out_ref[...] = jnp.dot(y, w2bd_ref[...], preferred_element_type=_f32) * dinv


def _mix2_call(agg1, xw1p, dinv, b1t, w2bd):
    return pl.pallas_call(
        _mix2_body,
        grid=(N // BN,),
        in_specs=[
            pl.BlockSpec((BN, 2 * HID), lambda i: (i, 0)),
            pl.BlockSpec((BN, 2 * HID), lambda i: (i, 0)),
            pl.BlockSpec((BN, 1), lambda i: (i, 0)),
            pl.BlockSpec((1, 2 * HID), lambda i: (0, 0)),
            pl.BlockSpec((2 * HID, 2 * OC), lambda i: (0, 0)),
        ],
        out_specs=pl.BlockSpec((BN, 2 * OC), lambda i: (i, 0)),
        out_shape=jax.ShapeDtypeStruct((N, 2 * OC), _f32),
    )(agg1, xw1p, dinv, b1t, w2bd)


def _fin_body(agga_ref, aggb_ref, xwp_ref, dinv_ref, b2t_ref, wf_ref,
              out_ref):
    agg = agga_ref[...] + aggb_ref[...]
    y = dinv_ref[...] * (agg + xwp_ref[...]) + b2t_ref[...]
    out_ref[...] = jnp.dot(y, wf_ref[...], preferred_element_type=_f32)


def _fin_call(agg2a, agg2b, xw2p, dinv, b2t, wf):
    return pl.pallas_call(
        _fin_body,
        grid=(N // BN,),
        in_specs=[
            pl.BlockSpec((BN, 2 * OC), lambda i: (i, 0)),
            pl.BlockSpec((BN, 2 * OC), lambda i: (i, 0)),
            pl.BlockSpec((BN, 2 * OC), lambda i: (i, 0)),
            pl.BlockSpec((BN, 1), lambda i: (i, 0)),
            pl.BlockSpec((1, 2 * OC), lambda i: (0, 0)),
            pl.BlockSpec((2 * OC, NB), lambda i: (0, 0)),
        ],
        out_specs=pl.BlockSpec((BN, NB), lambda i: (i, 0)),
        out_shape=jax.ShapeDtypeStruct((N, NB), _f32),
    )(agg2a, agg2b, xw2p, dinv, b2t, wf)


def kernel(x, edge_index, edge_weight, gru_Wih, gru_Whh, gru_bih, gru_bhh,
           conv1_W, conv1_b, conv2_W, conv2_b, fc_W, fc_b):
    x_fm = x.transpose(2, 0, 1).reshape(T, NB * N)
    x_fm = jnp.pad(x_fm, ((0, 0), (0, NPAD2 - NB * N)))
    h_fm = _gru_call(x_fm, gru_Wih[:, 0:1], gru_bih[:, None],
                     gru_Whh, gru_bhh[:, None])
    h0 = h_fm[:, :N].T
    h1 = h_fm[:, N:2 * N].T

    row = edge_index[0]
    col = edge_index[1]
    rowp = jnp.pad(row, (0, EP - E))
    colp = jnp.pad(col, (0, EP - E))
    ewp = jnp.pad(edge_weight, (0, EP - E))

    deg_parts = _deg_call(colp, ewp)
    degT = deg_parts.T

    xw1p, dinv = _mix1_call(h0, h1, degT, conv1_W)

    row2 = rowp.reshape(EP // CHUNK, CHUNK)
    col2 = colp.reshape(EP // CHUNK, CHUNK)
    ew2 = ewp.reshape(EP // CHUNK, CHUNK)

    z64 = jnp.zeros((RPT1, 2 * HID), _f32)
    agg1p = _agg_call(2 * HID, ACC1, RPT1, NBLK1, True,
                      xw1p, row2, col2, ew2, z64)
    agg1 = agg1p.reshape(NC, ACC1, 2 * HID)[:, :NHALF].reshape(N, 2 * HID)

    b1t = jnp.concatenate([conv1_b, conv1_b])[None, :]
    w2bd = jnp.zeros((2 * HID, 2 * OC), _f32)
    w2bd = w2bd.at[:HID, :OC].set(conv2_W).at[HID:, OC:].set(conv2_W)
    xw2p = _mix2_call(agg1, xw1p, dinv, b1t, w2bd)

    z32 = jnp.zeros((RPT2, 2 * OC), _f32)
    agg2p = _agg_call(2 * OC, ACC2, RPT2, NBLK2, False,
                      xw2p, row2, col2, ew2, z32)
    agg2a = agg2p[:N]
    agg2b = agg2p[ACC2:ACC2 + N]

    b2t = jnp.concatenate([conv2_b, conv2_b])[None, :]
    wf = jnp.zeros((2 * OC, NB), _f32)
    wf = wf.at[:OC, 0].set(fc_W[0]).at[OC:, 1].set(fc_W[0])
    y = _fin_call(agg2a, agg2b, xw2p, dinv, b2t, wf)

    return (y + fc_b).T


# speedup vs baseline: 25.7275x; 1.5246x over previous
import functools

import jax
import jax.numpy as jnp
from jax import lax
from jax.experimental import pallas as pl
from jax.experimental.pallas import tpu as pltpu
from jax.experimental.pallas import tpu_sc as plsc

N = 50000
E = 1600000
T = 24
HID = 32
G3 = 3 * HID
OC = 16
NB = 2

GCB = 2048
NPAD2 = 49 * GCB
BN = 2000

NC, NS = 2, 16
CHUNK = 128
BLK = 12
EP = 1622016
ACC = 50048
RPT = ACC // NS
NBLKT = 33
AGW = 32
EDGT_A = EP // (NC * NS)
CE = 6336

_f32 = jnp.float32


def _gru_body(x_ref, wih_ref, bih_ref, whh_ref, bhh_ref, h_ref):
    wih = wih_ref[...]
    bih = bih_ref[...]
    whh = whh_ref[...]
    bhh = bhh_ref[...]

    def step(t, h):
        xt = x_ref[pl.ds(t, 1), :]
        gi = wih * xt + bih
        gh = jnp.dot(whh, h, preferred_element_type=_f32) + bhh
        r = jax.nn.sigmoid(gi[:HID] + gh[:HID])
        z = jax.nn.sigmoid(gi[HID:2 * HID] + gh[HID:2 * HID])
        n = jnp.tanh(gi[2 * HID:] + r * gh[2 * HID:])
        return (1.0 - z) * n + z * h

    h_ref[...] = lax.fori_loop(0, T, step, jnp.zeros((HID, GCB), _f32))


def _gru_call(x_fm, wih, bih, whh, bhh):
    return pl.pallas_call(
        _gru_body,
        grid=(NPAD2 // GCB,),
        in_specs=[
            pl.BlockSpec((T, GCB), lambda i: (0, i)),
            pl.BlockSpec((G3, 1), lambda i: (0, 0)),
            pl.BlockSpec((G3, 1), lambda i: (0, 0)),
            pl.BlockSpec((G3, HID), lambda i: (0, 0)),
            pl.BlockSpec((G3, 1), lambda i: (0, 0)),
        ],
        out_specs=pl.BlockSpec((HID, GCB), lambda i: (0, i)),
        out_shape=jax.ShapeDtypeStruct((HID, NPAD2), _f32),
    )(x_fm, wih, bih, whh, bhh)


def _deg_body(col_hbm, ew_hbm, out_hbm, deg_v, col_v, ew_v):
    c = lax.axis_index("c")
    s = lax.axis_index("s")
    g = c * NS + s
    zero16 = jnp.zeros((16,), _f32)

    def zbody(i, carry):
        deg_v[pl.ds(i * 16, 16)] = zero16
        return carry

    lax.fori_loop(0, N // 16, zbody, 0)

    base = g * EDGT_A

    def chunk(k, carry):
        pltpu.sync_copy(col_hbm.at[pl.ds(base + k * CE, CE)], col_v)
        pltpu.sync_copy(ew_hbm.at[pl.ds(base + k * CE, CE)], ew_v)

        def inner(j, cc):
            idx = col_v[pl.ds(j * 16, 16)]
            w = ew_v[pl.ds(j * 16, 16)]
            plsc.addupdate_scatter(deg_v, [idx], w)
            return cc

        lax.fori_loop(0, CE // 16, inner, 0)
        return carry

    lax.fori_loop(0, EDGT_A // CE, chunk, 0)
    pltpu.sync_copy(deg_v, out_hbm.at[g])


def _deg_call(colp, ewp):
    fn = functools.partial(
        pl.kernel,
        out_type=jax.ShapeDtypeStruct((NC * NS, N), _f32),
        mesh=plsc.VectorSubcoreMesh(
            core_axis_name="c", subcore_axis_name="s",
            num_cores=NC, num_subcores=NS),
        scratch_types=[
            pltpu.VMEM((N,), _f32),
            pltpu.VMEM((CE,), jnp.int32),
            pltpu.VMEM((CE,), _f32),
        ],
        compiler_params=pltpu.CompilerParams(
            needs_layout_passes=False, use_tc_tiling_on_sc=False),
    )(_deg_body)
    return fn(colp, ewp)


def _agg_body(xw_hbm, row_hbm, col_hbm, ew_hbm, zeros_hbm, out_hbm,
              acc, rowblk, colblk, ewblk, r0, r1, r2,
              g0, g1, g2, s0, s1, s2):
    c_ax = lax.axis_index("c")
    s = lax.axis_index("s")
    rows = [r0, r1, r2]
    gsems = [g0, g1, g2]
    ssems = [s0, s1, s2]

    pltpu.sync_copy(zeros_hbm, acc.at[pl.ds(s * RPT, RPT)])
    plsc.subcore_barrier()

    tbase = (c_ax * NS + s) * (NBLKT * BLK)

    def gfire(ci, b):
        pltpu.async_copy(xw_hbm.at[rowblk.at[ci]], rows[b], gsems[b])

    def gwait(ci, b):
        pltpu.make_async_copy(
            xw_hbm.at[rowblk.at[ci]], rows[b], gsems[b]).wait()

    def sfire(ci, b):
        pltpu.async_copy(rows[b], acc.at[colblk.at[ci]], ssems[b],
                         add=True)

    def swait(b):
        pltpu.make_async_copy(
            rows[b], acc.at[colblk.at[0]], ssems[b]).wait()

    def blk_body(bi, carry):
        @pl.when(bi > 0)
        def _():
            for l in range(3):
                swait(l)

        cb = tbase + bi * BLK
        pltpu.sync_copy(row_hbm.at[pl.ds(cb, BLK)], rowblk)
        pltpu.sync_copy(col_hbm.at[pl.ds(cb, BLK)], colblk)
        pltpu.sync_copy(ew_hbm.at[pl.ds(cb, BLK)], ewblk)
        for l in range(2):
            gfire(l, l)

        def grp(g, cc):
            for l in range(3):
                c = g * 3 + l
                gwait(c, l)

                def scale(j16, sc_c):
                    w16 = ewblk[c, pl.ds(j16 * 16, 16)]
                    for ll in range(16):
                        w = w16[ll]
                        j = j16 * 16 + ll
                        for q in range(AGW // 16):
                            rows[l][j, pl.ds(q * 16, 16)] = (
                                rows[l][j, pl.ds(q * 16, 16)] * w)
                    return sc_c

                lax.fori_loop(0, CHUNK // 16, scale, 0)
                sfire(c, l)
                nb = (l + 2) % 3

                @pl.when(c + 2 < BLK)
                def _():
                    @pl.when(c > 0)
                    def _():
                        swait(nb)
                    gfire(c + 2, nb)
            return cc

        lax.fori_loop(0, BLK // 3, grp, 0)
        return carry

    lax.fori_loop(0, NBLKT, blk_body, 0)
    for l in range(3):
        swait(l)
    plsc.subcore_barrier()
    pltpu.sync_copy(acc.at[pl.ds(s * RPT, RPT)],
                    out_hbm.at[pl.ds(c_ax * ACC + s * RPT, RPT)])


def _agg_call(xw, row2, col2, ew2, zrows):
    fn = functools.partial(
        pl.kernel,
        out_type=jax.ShapeDtypeStruct((NC * ACC, AGW), _f32),
        mesh=plsc.VectorSubcoreMesh(
            core_axis_name="c", subcore_axis_name="s",
            num_cores=NC, num_subcores=NS),
        scratch_types=[
            pltpu.VMEM_SHARED((ACC, AGW), _f32),
            pltpu.VMEM((BLK, CHUNK), jnp.int32),
            pltpu.VMEM((BLK, CHUNK), jnp.int32),
            pltpu.VMEM((BLK, CHUNK), _f32),
            pltpu.VMEM((CHUNK, AGW), _f32),
            pltpu.VMEM((CHUNK, AGW), _f32),
            pltpu.VMEM((CHUNK, AGW), _f32),
            pltpu.SemaphoreType.DMA,
            pltpu.SemaphoreType.DMA,
            pltpu.SemaphoreType.DMA,
            pltpu.SemaphoreType.DMA,
            pltpu.SemaphoreType.DMA,
            pltpu.SemaphoreType.DMA,
        ],
        compiler_params=pltpu.CompilerParams(
            needs_layout_passes=False, use_tc_tiling_on_sc=False),
    )(_agg_body)
    return fn(xw, row2, col2, ew2, zrows)


def _mix1_body(h0_ref, h1_ref, degT_ref, w1_ref, xw_ref, dinv_ref):
    d = jnp.sum(degT_ref[...], axis=1, keepdims=True) + 1.0
    dinv = lax.rsqrt(d)
    w1 = w1_ref[...]
    a0 = jnp.dot(h0_ref[...], w1, preferred_element_type=_f32)
    a1 = jnp.dot(h1_ref[...], w1, preferred_element_type=_f32)
    xw_ref[...] = jnp.concatenate([a0, a1], axis=1) * dinv
    dinv_ref[...] = dinv


def _mix1_call(h0, h1, degT, w1):
    return pl.pallas_call(
        _mix1_body,
        grid=(N // BN,),
        in_specs=[
            pl.BlockSpec((BN, HID), lambda i: (i, 0)),
            pl.BlockSpec((BN, HID), lambda i: (i, 0)),
            pl.BlockSpec((BN, NC * NS), lambda i: (i, 0)),
            pl.BlockSpec((HID, HID), lambda i: (0, 0)),
        ],
        out_specs=[
            pl.BlockSpec((BN, 2 * HID), lambda i: (i, 0)),
            pl.BlockSpec((BN, 1), lambda i: (i, 0)),
        ],
        out_shape=[
            jax.ShapeDtypeStruct((N, 2 * HID), _f32),
            jax.ShapeDtypeStruct((N, 1), _f32),
        ],
    )(h0, h1, degT, w1)


def _mix2_body(p0a_ref, p0b_ref, p1a_ref, p1b_ref, xwp_ref, dinv_ref,
               b1t_ref, w2bd_ref, out_ref):
    agg = jnp.concatenate(
        [p0a_ref[...] + p0b_ref[...], p1a_ref[...] + p1b_ref[...]], axis=1)
    dinv = dinv_ref[...]
    y = jnp.maximum(dinv * (agg + xwp_ref[...]) + b1t_ref[...], 0.0)
    out_ref[...] = jnp.dot(y, w2bd_ref[...], preferred_element_type=_f32) * dinv


def _mix2_call(p0a, p0b, p1a, p1b, xw1p, dinv, b1t, w2bd):
    return pl.pallas_call(
        _mix2_body,
        grid=(N // BN,),
        in_specs=[
            pl.BlockSpec((BN, HID), lambda i: (i, 0)),
            pl.BlockSpec((BN, HID), lambda i: (i, 0)),
            pl.BlockSpec((BN, HID), lambda i: (i, 0)),
            pl.BlockSpec((BN, HID), lambda i: (i, 0)),
            pl.BlockSpec((BN, 2 * HID), lambda i: (i, 0)),
            pl.BlockSpec((BN, 1), lambda i: (i, 0)),
            pl.BlockSpec((1, 2 * HID), lambda i: (0, 0)),
            pl.BlockSpec((2 * HID, 2 * OC), lambda i: (0, 0)),
        ],
        out_specs=pl.BlockSpec((BN, 2 * OC), lambda i: (i, 0)),
        out_shape=jax.ShapeDtypeStruct((N, 2 * OC), _f32),
    )(p0a, p0b, p1a, p1b, xw1p, dinv, b1t, w2bd)


def _fin_body(agga_ref, aggb_ref, xwp_ref, dinv_ref, b2t_ref, wf_ref,
              out_ref):
    agg = agga_ref[...] + aggb_ref[...]
    y = dinv_ref[...] * (agg + xwp_ref[...]) + b2t_ref[...]
    out_ref[...] = jnp.dot(y, wf_ref[...], preferred_element_type=_f32)


def _fin_call(agg2a, agg2b, xw2p, dinv, b2t, wf):
    return pl.pallas_call(
        _fin_body,
        grid=(N // BN,),
        in_specs=[
            pl.BlockSpec((BN, 2 * OC), lambda i: (i, 0)),
            pl.BlockSpec((BN, 2 * OC), lambda i: (i, 0)),
            pl.BlockSpec((BN, 2 * OC), lambda i: (i, 0)),
            pl.BlockSpec((BN, 1), lambda i: (i, 0)),
            pl.BlockSpec((1, 2 * OC), lambda i: (0, 0)),
            pl.BlockSpec((2 * OC, NB), lambda i: (0, 0)),
        ],
        out_specs=pl.BlockSpec((BN, NB), lambda i: (i, 0)),
        out_shape=jax.ShapeDtypeStruct((N, NB), _f32),
    )(agg2a, agg2b, xw2p, dinv, b2t, wf)


def kernel(x, edge_index, edge_weight, gru_Wih, gru_Whh, gru_bih, gru_bhh,
           conv1_W, conv1_b, conv2_W, conv2_b, fc_W, fc_b):
    x_fm = x.transpose(2, 0, 1).reshape(T, NB * N)
    x_fm = jnp.pad(x_fm, ((0, 0), (0, NPAD2 - NB * N)))
    h_fm = _gru_call(x_fm, gru_Wih[:, 0:1], gru_bih[:, None],
                     gru_Whh, gru_bhh[:, None])
    h0 = h_fm[:, :N].T
    h1 = h_fm[:, N:2 * N].T

    row = edge_index[0]
    col = edge_index[1]
    rowp = jnp.pad(row, (0, EP - E))
    colp = jnp.pad(col, (0, EP - E))
    ewp = jnp.pad(edge_weight, (0, EP - E))

    deg_parts = _deg_call(colp, ewp)
    degT = deg_parts.T

    xw1p, dinv = _mix1_call(h0, h1, degT, conv1_W)

    row2 = rowp.reshape(EP // CHUNK, CHUNK)
    col2 = colp.reshape(EP // CHUNK, CHUNK)
    ew2 = ewp.reshape(EP // CHUNK, CHUNK)

    zrows = jnp.zeros((RPT, AGW), _f32)
    q0 = _agg_call(xw1p[:, :HID], row2, col2, ew2, zrows)
    q1 = _agg_call(xw1p[:, HID:], row2, col2, ew2, zrows)

    b1t = jnp.concatenate([conv1_b, conv1_b])[None, :]
    w2bd = jnp.zeros((2 * HID, 2 * OC), _f32)
    w2bd = w2bd.at[:HID, :OC].set(conv2_W).at[HID:, OC:].set(conv2_W)
    xw2p = _mix2_call(q0[:N], q0[ACC:ACC + N], q1[:N], q1[ACC:ACC + N],
                      xw1p, dinv, b1t, w2bd)

    agg2p = _agg_call(xw2p, row2, col2, ew2, zrows)
    agg2a = agg2p[:N]
    agg2b = agg2p[ACC:ACC + N]

    b2t = jnp.concatenate([conv2_b, conv2_b])[None, :]
    wf = jnp.zeros((2 * OC, NB), _f32)
    wf = wf.at[:OC, 0].set(fc_W[0]).at[OC:, 1].set(fc_W[0])
    y = _fin_call(agg2a, agg2b, xw2p, dinv, b2t, wf)

    return (y + fc_b).T


# speedup vs baseline: 26.4682x; 1.0288x over previous
import functools

import jax
import jax.numpy as jnp
from jax import lax
from jax.experimental import pallas as pl
from jax.experimental.pallas import tpu as pltpu
from jax.experimental.pallas import tpu_sc as plsc

N = 50000
E = 1600000
T = 24
HID = 32
G3 = 3 * HID
OC = 16
NB = 2

GCB = 2048
NPB = 25 * GCB
BN = 2000

NC, NS = 2, 16
CHUNK = 128
BLK = 12
EP = 1622016
ACC = 50048
RPT = ACC // NS
NBLKT = 33
AGW = 32
EDGT_A = EP // (NC * NS)
CE = 6336

_f32 = jnp.float32


def _gru_body(x_ref, wih_ref, bih_ref, whh_ref, bhh_ref, h_ref):
    wih = wih_ref[...]
    bih = bih_ref[...]
    whh = whh_ref[...]
    bhh = bhh_ref[...]

    def step(t, h):
        xt = x_ref[pl.ds(t, 1), :]
        gi = wih * xt + bih
        gh = jnp.dot(whh, h, preferred_element_type=_f32) + bhh
        r = jax.nn.sigmoid(gi[:HID] + gh[:HID])
        z = jax.nn.sigmoid(gi[HID:2 * HID] + gh[HID:2 * HID])
        n = jnp.tanh(gi[2 * HID:] + r * gh[2 * HID:])
        return (1.0 - z) * n + z * h

    h_ref[...] = lax.fori_loop(0, T, step, jnp.zeros((HID, GCB), _f32))


def _gru_call(x_fm, wih, bih, whh, bhh):
    return pl.pallas_call(
        _gru_body,
        grid=(NPB // GCB,),
        in_specs=[
            pl.BlockSpec((T, GCB), lambda i: (0, i)),
            pl.BlockSpec((G3, 1), lambda i: (0, 0)),
            pl.BlockSpec((G3, 1), lambda i: (0, 0)),
            pl.BlockSpec((G3, HID), lambda i: (0, 0)),
            pl.BlockSpec((G3, 1), lambda i: (0, 0)),
        ],
        out_specs=pl.BlockSpec((HID, GCB), lambda i: (0, i)),
        out_shape=jax.ShapeDtypeStruct((HID, NPB), _f32),
    )(x_fm, wih, bih, whh, bhh)


def _deg_body(col_hbm, ew_hbm, out_hbm, deg_v, col_v, ew_v):
    c = lax.axis_index("c")
    s = lax.axis_index("s")
    g = c * NS + s
    zero16 = jnp.zeros((16,), _f32)

    def zbody(i, carry):
        deg_v[pl.ds(i * 16, 16)] = zero16
        return carry

    lax.fori_loop(0, N // 16, zbody, 0)

    base = g * EDGT_A

    def chunk(k, carry):
        pltpu.sync_copy(col_hbm.at[pl.ds(base + k * CE, CE)], col_v)
        pltpu.sync_copy(ew_hbm.at[pl.ds(base + k * CE, CE)], ew_v)

        def inner(j, cc):
            idx = col_v[pl.ds(j * 16, 16)]
            w = ew_v[pl.ds(j * 16, 16)]
            plsc.addupdate_scatter(deg_v, [idx], w)
            return cc

        lax.fori_loop(0, CE // 16, inner, 0)
        return carry

    lax.fori_loop(0, EDGT_A // CE, chunk, 0)
    pltpu.sync_copy(deg_v, out_hbm.at[g])


def _deg_call(colp, ewp):
    fn = functools.partial(
        pl.kernel,
        out_type=jax.ShapeDtypeStruct((NC * NS, N), _f32),
        mesh=plsc.VectorSubcoreMesh(
            core_axis_name="c", subcore_axis_name="s",
            num_cores=NC, num_subcores=NS),
        scratch_types=[
            pltpu.VMEM((N,), _f32),
            pltpu.VMEM((CE,), jnp.int32),
            pltpu.VMEM((CE,), _f32),
        ],
        compiler_params=pltpu.CompilerParams(
            needs_layout_passes=False, use_tc_tiling_on_sc=False),
    )(_deg_body)
    return fn(colp, ewp)


def _agg_body(xw_hbm, row_hbm, col_hbm, ew_hbm, zeros_hbm, out_hbm,
              acc, rowblk, colblk, ewblk, r0, r1, r2,
              g0, g1, g2, s0, s1, s2):
    c_ax = lax.axis_index("c")
    s = lax.axis_index("s")
    rows = [r0, r1, r2]
    gsems = [g0, g1, g2]
    ssems = [s0, s1, s2]

    pltpu.sync_copy(zeros_hbm, acc.at[pl.ds(s * RPT, RPT)])
    plsc.subcore_barrier()

    tbase = (c_ax * NS + s) * (NBLKT * BLK)

    def gfire(ci, b):
        pltpu.async_copy(xw_hbm.at[rowblk.at[ci]], rows[b], gsems[b])

    def gwait(ci, b):
        pltpu.make_async_copy(
            xw_hbm.at[rowblk.at[ci]], rows[b], gsems[b]).wait()

    def sfire(ci, b):
        pltpu.async_copy(rows[b], acc.at[colblk.at[ci]], ssems[b],
                         add=True)

    def swait(b):
        pltpu.make_async_copy(
            rows[b], acc.at[colblk.at[0]], ssems[b]).wait()

    def blk_body(bi, carry):
        @pl.when(bi > 0)
        def _():
            for l in range(3):
                swait(l)

        cb = tbase + bi * BLK
        pltpu.sync_copy(row_hbm.at[pl.ds(cb, BLK)], rowblk)
        pltpu.sync_copy(col_hbm.at[pl.ds(cb, BLK)], colblk)
        pltpu.sync_copy(ew_hbm.at[pl.ds(cb, BLK)], ewblk)
        for l in range(2):
            gfire(l, l)

        def grp(g, cc):
            for l in range(3):
                c = g * 3 + l
                gwait(c, l)

                def scale(j16, sc_c):
                    w16 = ewblk[c, pl.ds(j16 * 16, 16)]
                    for ll in range(16):
                        w = w16[ll]
                        j = j16 * 16 + ll
                        for q in range(AGW // 16):
                            rows[l][j, pl.ds(q * 16, 16)] = (
                                rows[l][j, pl.ds(q * 16, 16)] * w)
                    return sc_c

                lax.fori_loop(0, CHUNK // 16, scale, 0)
                sfire(c, l)
                nb = (l + 2) % 3

                @pl.when(c + 2 < BLK)
                def _():
                    @pl.when(c > 0)
                    def _():
                        swait(nb)
                    gfire(c + 2, nb)
            return cc

        lax.fori_loop(0, BLK // 3, grp, 0)
        return carry

    lax.fori_loop(0, NBLKT, blk_body, 0)
    for l in range(3):
        swait(l)
    plsc.subcore_barrier()
    pltpu.sync_copy(acc.at[pl.ds(s * RPT, RPT)],
                    out_hbm.at[pl.ds(c_ax * ACC + s * RPT, RPT)])


def _agg_call(xw, row2, col2, ew2, zrows):
    fn = functools.partial(
        pl.kernel,
        out_type=jax.ShapeDtypeStruct((NC * ACC, AGW), _f32),
        mesh=plsc.VectorSubcoreMesh(
            core_axis_name="c", subcore_axis_name="s",
            num_cores=NC, num_subcores=NS),
        scratch_types=[
            pltpu.VMEM_SHARED((ACC, AGW), _f32),
            pltpu.VMEM((BLK, CHUNK), jnp.int32),
            pltpu.VMEM((BLK, CHUNK), jnp.int32),
            pltpu.VMEM((BLK, CHUNK), _f32),
            pltpu.VMEM((CHUNK, AGW), _f32),
            pltpu.VMEM((CHUNK, AGW), _f32),
            pltpu.VMEM((CHUNK, AGW), _f32),
            pltpu.SemaphoreType.DMA,
            pltpu.SemaphoreType.DMA,
            pltpu.SemaphoreType.DMA,
            pltpu.SemaphoreType.DMA,
            pltpu.SemaphoreType.DMA,
            pltpu.SemaphoreType.DMA,
        ],
        compiler_params=pltpu.CompilerParams(
            needs_layout_passes=False, use_tc_tiling_on_sc=False),
    )(_agg_body)
    return fn(xw, row2, col2, ew2, zrows)


def _mix1a_body(h0_ref, degT_ref, w1_ref, xw_ref, dinv_ref):
    d = jnp.sum(degT_ref[...], axis=1, keepdims=True) + 1.0
    dinv = lax.rsqrt(d)
    xw_ref[...] = jnp.dot(h0_ref[...], w1_ref[...],
                          preferred_element_type=_f32) * dinv
    dinv_ref[...] = dinv


def _mix1a_call(h0, degT, w1):
    return pl.pallas_call(
        _mix1a_body,
        grid=(N // BN,),
        in_specs=[
            pl.BlockSpec((BN, HID), lambda i: (i, 0)),
            pl.BlockSpec((BN, NC * NS), lambda i: (i, 0)),
            pl.BlockSpec((HID, HID), lambda i: (0, 0)),
        ],
        out_specs=[
            pl.BlockSpec((BN, HID), lambda i: (i, 0)),
            pl.BlockSpec((BN, 1), lambda i: (i, 0)),
        ],
        out_shape=[
            jax.ShapeDtypeStruct((N, HID), _f32),
            jax.ShapeDtypeStruct((N, 1), _f32),
        ],
    )(h0, degT, w1)


def _mix1b_body(h1_ref, dinv_ref, w1_ref, xw_ref):
    xw_ref[...] = jnp.dot(h1_ref[...], w1_ref[...],
                          preferred_element_type=_f32) * dinv_ref[...]


def _mix1b_call(h1, dinv, w1):
    return pl.pallas_call(
        _mix1b_body,
        grid=(N // BN,),
        in_specs=[
            pl.BlockSpec((BN, HID), lambda i: (i, 0)),
            pl.BlockSpec((BN, 1), lambda i: (i, 0)),
            pl.BlockSpec((HID, HID), lambda i: (0, 0)),
        ],
        out_specs=pl.BlockSpec((BN, HID), lambda i: (i, 0)),
        out_shape=jax.ShapeDtypeStruct((N, HID), _f32),
    )(h1, dinv, w1)


def _mix2_body(p0a_ref, p0b_ref, p1a_ref, p1b_ref, xwa_ref, xwb_ref,
               dinv_ref, b1t_ref, w2bd_ref, out_ref):
    agg = jnp.concatenate(
        [p0a_ref[...] + p0b_ref[...], p1a_ref[...] + p1b_ref[...]], axis=1)
    xwp = jnp.concatenate([xwa_ref[...], xwb_ref[...]], axis=1)
    dinv = dinv_ref[...]
    y = jnp.maximum(dinv * (agg + xwp) + b1t_ref[...], 0.0)
    out_ref[...] = jnp.dot(y, w2bd_ref[...], preferred_element_type=_f32) * dinv


def _mix2_call(p0a, p0b, p1a, p1b, xw1a, xw1b, dinv, b1t, w2bd):
    return pl.pallas_call(
        _mix2_body,
        grid=(N // BN,),
        in_specs=[
            pl.BlockSpec((BN, HID), lambda i: (i, 0)),
            pl.BlockSpec((BN, HID), lambda i: (i, 0)),
            pl.BlockSpec((BN, HID), lambda i: (i, 0)),
            pl.BlockSpec((BN, HID), lambda i: (i, 0)),
            pl.BlockSpec((BN, HID), lambda i: (i, 0)),
            pl.BlockSpec((BN, HID), lambda i: (i, 0)),
            pl.BlockSpec((BN, 1), lambda i: (i, 0)),
            pl.BlockSpec((1, 2 * HID), lambda i: (0, 0)),
            pl.BlockSpec((2 * HID, 2 * OC), lambda i: (0, 0)),
        ],
        out_specs=pl.BlockSpec((BN, 2 * OC), lambda i: (i, 0)),
        out_shape=jax.ShapeDtypeStruct((N, 2 * OC), _f32),
    )(p0a, p0b, p1a, p1b, xw1a, xw1b, dinv, b1t, w2bd)


def _fin_body(agga_ref, aggb_ref, xwp_ref, dinv_ref, b2t_ref, wf_ref,
              out_ref):
    agg = agga_ref[...] + aggb_ref[...]
    y = dinv_ref[...] * (agg + xwp_ref[...]) + b2t_ref[...]
    out_ref[...] = jnp.dot(y, wf_ref[...], preferred_element_type=_f32)


def _fin_call(agg2a, agg2b, xw2p, dinv, b2t, wf):
    return pl.pallas_call(
        _fin_body,
        grid=(N // BN,),
        in_specs=[
            pl.BlockSpec((BN, 2 * OC), lambda i: (i, 0)),
            pl.BlockSpec((BN, 2 * OC), lambda i: (i, 0)),
            pl.BlockSpec((BN, 2 * OC), lambda i: (i, 0)),
            pl.BlockSpec((BN, 1), lambda i: (i, 0)),
            pl.BlockSpec((1, 2 * OC), lambda i: (0, 0)),
            pl.BlockSpec((2 * OC, NB), lambda i: (0, 0)),
        ],
        out_specs=pl.BlockSpec((BN, NB), lambda i: (i, 0)),
        out_shape=jax.ShapeDtypeStruct((N, NB), _f32),
    )(agg2a, agg2b, xw2p, dinv, b2t, wf)


def kernel(x, edge_index, edge_weight, gru_Wih, gru_Whh, gru_bih, gru_bhh,
           conv1_W, conv1_b, conv2_W, conv2_b, fc_W, fc_b):
    wih = gru_Wih[:, 0:1]
    bih = gru_bih[:, None]
    bhh = gru_bhh[:, None]
    pad = ((0, 0), (0, NPB - N))
    h_fm0 = _gru_call(jnp.pad(x[0].T, pad), wih, bih, gru_Whh, bhh)
    h_fm1 = _gru_call(jnp.pad(x[1].T, pad), wih, bih, gru_Whh, bhh)
    h0 = h_fm0[:, :N].T
    h1 = h_fm1[:, :N].T

    row = edge_index[0]
    col = edge_index[1]
    rowp = jnp.pad(row, (0, EP - E))
    colp = jnp.pad(col, (0, EP - E))
    ewp = jnp.pad(edge_weight, (0, EP - E))

    deg_parts = _deg_call(colp, ewp)
    degT = deg_parts.T

    xw1a, dinv = _mix1a_call(h0, degT, conv1_W)

    row2 = rowp.reshape(EP // CHUNK, CHUNK)
    col2 = colp.reshape(EP // CHUNK, CHUNK)
    ew2 = ewp.reshape(EP // CHUNK, CHUNK)

    zrows = jnp.zeros((RPT, AGW), _f32)
    q0 = _agg_call(xw1a, row2, col2, ew2, zrows)
    xw1b = _mix1b_call(h1, dinv, conv1_W)
    q1 = _agg_call(xw1b, row2, col2, ew2, zrows)

    b1t = jnp.concatenate([conv1_b, conv1_b])[None, :]
    w2bd = jnp.zeros((2 * HID, 2 * OC), _f32)
    w2bd = w2bd.at[:HID, :OC].set(conv2_W).at[HID:, OC:].set(conv2_W)
    xw2p = _mix2_call(q0[:N], q0[ACC:ACC + N], q1[:N], q1[ACC:ACC + N],
                      xw1a, xw1b, dinv, b1t, w2bd)

    agg2p = _agg_call(xw2p, row2, col2, ew2, zrows)
    agg2a = agg2p[:N]
    agg2b = agg2p[ACC:ACC + N]

    b2t = jnp.concatenate([conv2_b, conv2_b])[None, :]
    wf = jnp.zeros((2 * OC, NB), _f32)
    wf = wf.at[:OC, 0].set(fc_W[0]).at[OC:, 1].set(fc_W[0])
    y = _fin_call(agg2a, agg2b, xw2p, dinv, b2t, wf)

    return (y + fc_b).T
